# baseline (speedup 1.0000x reference)
import jax
import jax.numpy as jnp
from jax import lax
from jax.experimental import pallas as pl
from jax.experimental.pallas import tpu as pltpu
from jax.experimental.pallas import tpu_sc as plsc

N = 10000
E = 320000
D = 128
NC = 2
NS = 16
L = 16
NW = NC * NS
EW = E // NW
G = 80
NG = EW // G
RSTRIDE = 624
RROWS = 640

NP = 10240
BR = 512
GRID = NP // BR


def _project_body(x_ref, wi_ref, b_ref, w_ref, a1_ref, a2_ref,
                  z_ref, s1_ref, s2_ref):
    x = x_ref[...]
    h = jax.nn.gelu(
        lax.dot_general(x, wi_ref[...], (((1,), (1,)), ((), ())),
                        preferred_element_type=jnp.float32)
        + b_ref[...])
    z = lax.dot_general(h, w_ref[...], (((1,), (1,)), ((), ())),
                        preferred_element_type=jnp.float32)
    z_ref[...] = z
    i = pl.program_id(0)
    s1_ref[pl.ds(i * BR, BR)] = jnp.sum(z * a1_ref[...], axis=1)
    s2_ref[pl.ds(i * BR, BR)] = jnp.sum(z * a2_ref[...], axis=1)


_project = pl.pallas_call(
    _project_body,
    grid=(GRID,),
    in_specs=[
        pl.BlockSpec((BR, D), lambda i: (i, 0)),
        pl.BlockSpec((D, D), lambda i: (0, 0)),
        pl.BlockSpec((D,), lambda i: (0,)),
        pl.BlockSpec((D, D), lambda i: (0, 0)),
        pl.BlockSpec((1, D), lambda i: (0, 0)),
        pl.BlockSpec((1, D), lambda i: (0, 0)),
    ],
    out_specs=[
        pl.BlockSpec((BR, D), lambda i: (i, 0)),
        pl.BlockSpec((NP,), lambda i: (0,)),
        pl.BlockSpec((NP,), lambda i: (0,)),
    ],
    out_shape=[
        jax.ShapeDtypeStruct((NP, D), jnp.float32),
        jax.ShapeDtypeStruct((NP,), jnp.float32),
        jax.ShapeDtypeStruct((NP,), jnp.float32),
    ],
)


def _edge_body(z_hbm, s1_hbm, s2_hbm, src_hbm, dst_hbm, u_hbm, d_hbm,
               rows0, rows1, rows2, sa0, sa1, sa2, sb0, sb1, sb2,
               w0, w1, w2, sc0, sc1, sc2, dc0, dc1, dc2, zb_v, u_sh, den_sh,
               gsem0, gsem1, gsem2, ssem0, ssem1, ssem2,
               isem0, isem1, isem2):
    cid = lax.axis_index("c")
    sid = lax.axis_index("s")
    wid = sid * NC + cid

    rows = [rows0, rows1, rows2]
    sa = [sa0, sa1, sa2]
    sb = [sb0, sb1, sb2]
    wv = [w0, w1, w2]
    sc = [sc0, sc1, sc2]
    dc = [dc0, dc1, dc2]
    gsem = [gsem0, gsem1, gsem2]
    ssem = [ssem0, ssem1, ssem2]
    isem = [isem0, isem1, isem2]

    def issue_idx(g, b):
        off = wid * EW + g * G
        pltpu.async_copy(src_hbm.at[pl.ds(off, G)], sc[b], isem[b])
        pltpu.async_copy(dst_hbm.at[pl.ds(off, G)], dc[b], isem[b])

    def wait_idx(g, b):
        off = wid * EW + g * G
        pltpu.make_async_copy(src_hbm.at[pl.ds(off, G)], sc[b], isem[b]).wait()
        pltpu.make_async_copy(dst_hbm.at[pl.ds(off, G)], dc[b], isem[b]).wait()

    def issue_gather(b):
        pltpu.async_copy(z_hbm.at[sc[b]], rows[b], gsem[b])
        pltpu.async_copy(s1_hbm.at[sc[b]], sa[b], isem[b])
        pltpu.async_copy(s2_hbm.at[dc[b]], sb[b], isem[b])

    def wait_gather_scalars(b):
        pltpu.make_async_copy(s1_hbm.at[sc[b]], sa[b], isem[b]).wait()
        pltpu.make_async_copy(s2_hbm.at[dc[b]], sb[b], isem[b]).wait()

    def wait_gather_rows(b):
        pltpu.make_async_copy(z_hbm.at[sc[b]], rows[b], gsem[b]).wait()

    def issue_scatter(b):
        pltpu.async_copy(rows[b], u_sh.at[dc[b]], ssem[b], add=True)
        pltpu.async_copy(wv[b], den_sh.at[dc[b]], ssem[b], add=True)

    def wait_scatter(b):
        pltpu.make_async_copy(rows[b], u_sh.at[dc[b]], ssem[b]).wait()
        pltpu.make_async_copy(wv[b], den_sh.at[dc[b]], ssem[b]).wait()

    def compute(b):
        wait_gather_scalars(b)

        @plsc.parallel_loop(0, G // L, unroll=G // L)
        def scal(i):
            e = sa[b][pl.ds(i * L, L)] + sb[b][pl.ds(i * L, L)]
            e = jnp.where(e >= 0, e, e * jnp.float32(0.01))
            wv[b][pl.ds(i * L, L)] = jnp.exp(e)

        wait_gather_rows(b)

        @plsc.parallel_loop(0, G, unroll=4)
        def scale(r):
            wspl = plsc.load_gather(wv[b], [jnp.full((L,), r, jnp.int32)])
            for j in range(D // L):
                rows[b][r, pl.ds(j * L, L)] = (
                    rows[b][r, pl.ds(j * L, L)] * wspl)

    issue_idx(0, 0)
    issue_idx(1, 1)

    zeros = jnp.zeros((L,), jnp.float32)

    def zero_zb(i, _):
        for j in range(4):
            zb_v[pl.ds((i * 4 + j) * L, L)] = zeros
        return 0
    lax.fori_loop(0, RROWS // (L * 4), zero_zb, 0)

    def zero_rows(r, _):
        for j in range(D // L):
            rows2[r, pl.ds(j * L, L)] = zeros
        return 0
    lax.fori_loop(0, G, zero_rows, 0)

    base = sid * RSTRIDE
    for t in range(RROWS // G):
        pltpu.async_copy(rows2, u_sh.at[pl.ds(base + t * G, G)], gsem[t % 3])
    pltpu.sync_copy(zb_v, den_sh.at[pl.ds(base, RROWS)])
    for t in range(RROWS // G):
        pltpu.make_async_copy(rows2, u_sh.at[pl.ds(base + t * G, G)],
                              gsem[t % 3]).wait()
    plsc.subcore_barrier()

    wait_idx(0, 0)
    issue_gather(0)
    wait_idx(1, 1)
    issue_gather(1)

    def slot(g, b, first=False, ahead=True):
        b2 = (b + 2) % 3
        compute(b)
        issue_scatter(b)
        if not first:
            wait_scatter(b2)

        def fetch_ahead():
            issue_idx(g + 2, b2)
            wait_idx(g + 2, b2)
            issue_gather(b2)
        if ahead is True:
            fetch_ahead()
        elif ahead is not False:
            pl.when(ahead)(fetch_ahead)

    slot(0, 0, first=True)

    def triple(t, _):
        for c in range(3):
            g = 3 * t + 1 + c
            slot(g, (1 + c) % 3, ahead=(g <= NG - 3))
        return 0
    lax.fori_loop(0, (NG - 2) // 3, triple, 0)

    slot(NG - 1, (NG - 1) % 3, ahead=False)
    wait_scatter((NG - 1) % 3)

    plsc.subcore_barrier()
    for t in range(RROWS // G):
        pltpu.async_copy(u_sh.at[pl.ds(base + t * G, G)],
                         u_hbm.at[cid, pl.ds(base + t * G, G)],
                         gsem[t % 3])
    pltpu.sync_copy(den_sh.at[pl.ds(base, RROWS)], zb_v)
    pltpu.sync_copy(zb_v, d_hbm.at[pl.ds(cid * NP + base, RROWS)])
    for t in range(RROWS // G):
        pltpu.make_async_copy(u_sh.at[pl.ds(base + t * G, G)],
                              u_hbm.at[cid, pl.ds(base + t * G, G)],
                              gsem[t % 3]).wait()


_edge = pl.kernel(
    _edge_body,
    out_type=(
        jax.ShapeDtypeStruct((NC, NP, D), jnp.float32),
        jax.ShapeDtypeStruct((NC * NP,), jnp.float32),
    ),
    mesh=plsc.VectorSubcoreMesh(core_axis_name="c", subcore_axis_name="s",
                                num_cores=NC, num_subcores=NS),
    compiler_params=pltpu.CompilerParams(needs_layout_passes=False),
    scratch_types=(
        [pltpu.VMEM((G, D), jnp.float32)] * 3
        + [pltpu.VMEM((G,), jnp.float32)] * 9
        + [pltpu.VMEM((G,), jnp.int32)] * 6
        + [pltpu.VMEM((RROWS,), jnp.float32)]
        + [pltpu.VMEM_SHARED((N, D), jnp.float32),
           pltpu.VMEM_SHARED((N,), jnp.float32)]
        + [pltpu.SemaphoreType.DMA] * 9
    ),
)


def _combine_body(u_ref, d_ref, o_ref):
    u = u_ref[0] + u_ref[1]
    den = d_ref[0] + d_ref[1]
    o_ref[...] = u * (1.0 / jnp.maximum(den, 1e-16))[:, None]


_combine = pl.pallas_call(
    _combine_body,
    grid=(GRID,),
    in_specs=[
        pl.BlockSpec((NC, BR, D), lambda i: (0, i, 0)),
        pl.BlockSpec((NC, BR), lambda i: (0, i)),
    ],
    out_specs=pl.BlockSpec((BR, D), lambda i: (i, 0)),
    out_shape=jax.ShapeDtypeStruct((N, D), jnp.float32),
)


def kernel(x, edge_index, W_inp, b_inp, W, attn):
    a1 = attn[:, :D]
    a2 = attn[:, D:]
    x_pad = jnp.pad(x, ((0, NP - N), (0, 0)))
    z, s1, s2 = _project(x_pad, W_inp, b_inp, W, a1, a2)
    u, den = _edge(z, s1, s2, edge_index[0], edge_index[1])
    return _combine(u, den.reshape(NC, NP))


# speedup vs baseline: 1.0127x; 1.0127x over previous
import jax
import jax.numpy as jnp
from jax import lax
from jax.experimental import pallas as pl
from jax.experimental.pallas import tpu as pltpu
from jax.experimental.pallas import tpu_sc as plsc

N = 10000
E = 320000
D = 128
NC = 2
NS = 16
L = 16
NW = NC * NS
EW = E // NW
G = 80
NG = EW // G
RSTRIDE = 624
RROWS = 640

NP = 10240
BR = 512
GRID = NP // BR


def _project_body(x_ref, wi_ref, b_ref, w_ref, a1_ref, a2_ref,
                  z_ref, s1_ref, s2_ref):
    x = x_ref[...]
    h = jax.nn.gelu(
        lax.dot_general(x, wi_ref[...], (((1,), (1,)), ((), ())),
                        preferred_element_type=jnp.float32)
        + b_ref[...])
    z = lax.dot_general(h, w_ref[...], (((1,), (1,)), ((), ())),
                        preferred_element_type=jnp.float32)
    z_ref[...] = z
    i = pl.program_id(0)
    s1_ref[pl.ds(i * BR, BR)] = jnp.sum(z * a1_ref[...], axis=1)
    s2_ref[pl.ds(i * BR, BR)] = jnp.sum(z * a2_ref[...], axis=1)


_project = pl.pallas_call(
    _project_body,
    grid=(GRID,),
    in_specs=[
        pl.BlockSpec((BR, D), lambda i: (i, 0)),
        pl.BlockSpec((D, D), lambda i: (0, 0)),
        pl.BlockSpec((D,), lambda i: (0,)),
        pl.BlockSpec((D, D), lambda i: (0, 0)),
        pl.BlockSpec((1, D), lambda i: (0, 0)),
        pl.BlockSpec((1, D), lambda i: (0, 0)),
    ],
    out_specs=[
        pl.BlockSpec((BR, D), lambda i: (i, 0)),
        pl.BlockSpec((NP,), lambda i: (0,)),
        pl.BlockSpec((NP,), lambda i: (0,)),
    ],
    out_shape=[
        jax.ShapeDtypeStruct((NP, D), jnp.float32),
        jax.ShapeDtypeStruct((NP,), jnp.float32),
        jax.ShapeDtypeStruct((NP,), jnp.float32),
    ],
)


def _edge_body(z_hbm, s1_hbm, s2_hbm, src_hbm, dst_hbm, u_hbm, d_hbm,
               rows0, rows1, rows2, sa0, sa1, sa2, sb0, sb1, sb2,
               w0, w1, w2, sc0, sc1, sc2, dc0, dc1, dc2, zb_v, u_sh, den_sh,
               gsem0, gsem1, gsem2, ssem0, ssem1, ssem2,
               isem0, isem1, isem2):
    cid = lax.axis_index("c")
    sid = lax.axis_index("s")
    wid = sid * NC + cid

    rows = [rows0, rows1, rows2]
    sa = [sa0, sa1, sa2]
    sb = [sb0, sb1, sb2]
    wv = [w0, w1, w2]
    sc = [sc0, sc1, sc2]
    dc = [dc0, dc1, dc2]
    gsem = [gsem0, gsem1, gsem2]
    ssem = [ssem0, ssem1, ssem2]
    isem = [isem0, isem1, isem2]

    def issue_idx(g, b):
        off = wid * EW + g * G
        pltpu.async_copy(src_hbm.at[pl.ds(off, G)], sc[b], isem[b])
        pltpu.async_copy(dst_hbm.at[pl.ds(off, G)], dc[b], isem[b])

    def wait_idx(g, b):
        off = wid * EW + g * G
        pltpu.make_async_copy(src_hbm.at[pl.ds(off, G)], sc[b], isem[b]).wait()
        pltpu.make_async_copy(dst_hbm.at[pl.ds(off, G)], dc[b], isem[b]).wait()

    def issue_gather(b):
        pltpu.async_copy(z_hbm.at[sc[b]], rows[b], gsem[b])
        pltpu.async_copy(s1_hbm.at[sc[b]], sa[b], isem[b])
        pltpu.async_copy(s2_hbm.at[dc[b]], sb[b], isem[b])

    def wait_gather_scalars(b):
        pltpu.make_async_copy(s1_hbm.at[sc[b]], sa[b], isem[b]).wait()
        pltpu.make_async_copy(s2_hbm.at[dc[b]], sb[b], isem[b]).wait()

    def wait_gather_rows(b):
        pltpu.make_async_copy(z_hbm.at[sc[b]], rows[b], gsem[b]).wait()

    def issue_scatter(b):
        pltpu.async_copy(rows[b], u_sh.at[dc[b]], ssem[b], add=True)
        pltpu.async_copy(wv[b], den_sh.at[dc[b]], ssem[b], add=True)

    def wait_scatter(b):
        pltpu.make_async_copy(rows[b], u_sh.at[dc[b]], ssem[b]).wait()
        pltpu.make_async_copy(wv[b], den_sh.at[dc[b]], ssem[b]).wait()

    def compute(b):
        wait_gather_scalars(b)

        @plsc.parallel_loop(0, G // L, unroll=G // L)
        def scal(i):
            e = sa[b][pl.ds(i * L, L)] + sb[b][pl.ds(i * L, L)]
            e = jnp.where(e >= 0, e, e * jnp.float32(0.01))
            wv[b][pl.ds(i * L, L)] = jnp.exp(e)

        wait_gather_rows(b)

        @plsc.parallel_loop(0, G, unroll=4)
        def scale(r):
            wspl = plsc.load_gather(wv[b], [jnp.full((L,), r, jnp.int32)])
            for j in range(D // L):
                rows[b][r, pl.ds(j * L, L)] = (
                    rows[b][r, pl.ds(j * L, L)] * wspl)

    issue_idx(0, 0)
    issue_idx(1, 1)

    zeros = jnp.zeros((L,), jnp.float32)

    def zero_zb(i, _):
        for j in range(4):
            zb_v[pl.ds((i * 4 + j) * L, L)] = zeros
        return 0
    lax.fori_loop(0, RROWS // (L * 4), zero_zb, 0)

    def zero_rows(r, _):
        for j in range(D // L):
            rows2[r, pl.ds(j * L, L)] = zeros
        return 0
    lax.fori_loop(0, G, zero_rows, 0)

    base = sid * RSTRIDE
    for t in range(RROWS // G):
        pltpu.async_copy(rows2, u_sh.at[pl.ds(base + t * G, G)], gsem[t % 3])
    pltpu.sync_copy(zb_v, den_sh.at[pl.ds(base, RROWS)])
    for t in range(RROWS // G):
        pltpu.make_async_copy(rows2, u_sh.at[pl.ds(base + t * G, G)],
                              gsem[t % 3]).wait()
    plsc.subcore_barrier()

    wait_idx(0, 0)
    issue_gather(0)
    wait_idx(1, 1)
    issue_gather(1)

    def slot(g, b, first=False, ahead=True):
        b2 = (b + 2) % 3
        compute(b)
        issue_scatter(b)
        if not first:
            wait_scatter(b2)

        def fetch_ahead():
            issue_idx(g + 2, b2)
            wait_idx(g + 2, b2)
            issue_gather(b2)
        if ahead is True:
            fetch_ahead()
        elif ahead is not False:
            pl.when(ahead)(fetch_ahead)

    slot(0, 0, first=True)

    def triple(t, _):
        for c in range(3):
            g = 3 * t + 1 + c
            slot(g, (1 + c) % 3, ahead=(g <= NG - 3))
        return 0
    lax.fori_loop(0, (NG - 2) // 3, triple, 0)

    slot(NG - 1, (NG - 1) % 3, ahead=False)
    wait_scatter((NG - 1) % 3)

    plsc.subcore_barrier()
    for t in range(RROWS // G):
        pltpu.async_copy(u_sh.at[pl.ds(base + t * G, G)],
                         u_hbm.at[cid, pl.ds(base + t * G, G)],
                         gsem[t % 3])
    pltpu.sync_copy(den_sh.at[pl.ds(base, RROWS)], zb_v)
    pltpu.sync_copy(zb_v, d_hbm.at[pl.ds(cid * NP + base, RROWS)])
    for t in range(RROWS // G):
        pltpu.make_async_copy(u_sh.at[pl.ds(base + t * G, G)],
                              u_hbm.at[cid, pl.ds(base + t * G, G)],
                              gsem[t % 3]).wait()


_edge = pl.kernel(
    _edge_body,
    out_type=(
        jax.ShapeDtypeStruct((NC, NP, D), jnp.float32),
        jax.ShapeDtypeStruct((NC * NP,), jnp.float32),
    ),
    mesh=plsc.VectorSubcoreMesh(core_axis_name="c", subcore_axis_name="s",
                                num_cores=NC, num_subcores=NS),
    compiler_params=pltpu.CompilerParams(needs_layout_passes=False),
    scratch_types=(
        [pltpu.VMEM((G, D), jnp.float32)] * 3
        + [pltpu.VMEM((G,), jnp.float32)] * 9
        + [pltpu.VMEM((G,), jnp.int32)] * 6
        + [pltpu.VMEM((RROWS,), jnp.float32)]
        + [pltpu.VMEM_SHARED((N, D), jnp.float32),
           pltpu.VMEM_SHARED((N,), jnp.float32)]
        + [pltpu.SemaphoreType.DMA] * 9
    ),
)


def _combine_body(u_ref, d_ref, o_ref):
    u = u_ref[0] + u_ref[1]
    den = d_ref[0] + d_ref[1]
    o_ref[...] = u * (1.0 / jnp.maximum(den, 1e-16))[:, None]


_combine = pl.pallas_call(
    _combine_body,
    grid=(GRID,),
    in_specs=[
        pl.BlockSpec((NC, BR, D), lambda i: (0, i, 0)),
        pl.BlockSpec((NC, BR), lambda i: (0, i)),
    ],
    out_specs=pl.BlockSpec((BR, D), lambda i: (i, 0)),
    out_shape=jax.ShapeDtypeStruct((N, D), jnp.float32),
)


def kernel(x, edge_index, W_inp, b_inp, W, attn):
    a1 = attn[:, :D]
    a2 = attn[:, D:]
    z, s1, s2 = _project(x, W_inp, b_inp, W, a1, a2)
    u, den = _edge(z, s1, s2, edge_index[0], edge_index[1])
    return _combine(u, den.reshape(NC, NP))
